# Initial kernel scaffold; baseline (speedup 1.0000x reference)
import jax
import jax.numpy as jnp
from jax.experimental import pallas as pl


def kernel(x, edge_index, W_self1, W_neigh1, b1, W_self2, W_neigh2, b2):
    raise NotImplementedError("write your pallas kernel here")


# speedup vs baseline: 5.9170x; 5.9170x over previous
import jax
import jax.numpy as jnp
from jax import lax
from jax.experimental import pallas as pl
from jax.experimental.pallas import tpu as pltpu
from jax.experimental.pallas import tpu_sc as plsc

N_NODES = 10000
N_EDGES = 320000
D = 128
NS = 16
NC = 2
N_PAD = 10240
E_PAD = 327680
ROWS_PER_TILE = E_PAD // (NC * NS) // 128
CH_ROWS = 1
N_CHUNKS = ROWS_PER_TILE // CH_ROWS
ROWS_OUT = N_PAD // NS // 128


def _make_sc_agg(with_deg: bool):
    mesh = plsc.VectorSubcoreMesh(core_axis_name="c", subcore_axis_name="s")
    out_type = [
        jax.ShapeDtypeStruct((NC, N_PAD, D), jnp.float32),
    ]
    if with_deg:
        out_type.append(
            jax.ShapeDtypeStruct((NC, N_PAD, D), jnp.float32))
    scratch = [
        pltpu.VMEM((CH_ROWS * 128,), jnp.int32),
        pltpu.VMEM((CH_ROWS * 128,), jnp.int32),
        pltpu.VMEM((CH_ROWS * 128, D), jnp.float32),
        pltpu.VMEM_SHARED((N_PAD, D), jnp.float32),
        pltpu.SemaphoreType.DMA,
    ]
    nch = CH_ROWS * 128

    def body(table, srcb, dstb, *rest):
        if with_deg:
            (agg_out, deg_out, src_v, dst_v, rows_v, agg_s, gsem) = rest
        else:
            (agg_out, src_v, dst_v, rows_v, agg_s, gsem) = rest
        c = lax.axis_index("c")
        s = lax.axis_index("s")

        def fill(val):
            v16 = jnp.full((16,), val, jnp.float32)

            def f(r, carry):
                for j in range(D // 16):
                    rows_v[r, pl.ds(j * 16, 16)] = v16
                return carry
            lax.fori_loop(0, 128, f, 0)

        def zero_agg():
            def zcp(k, carry):
                pltpu.sync_copy(rows_v.at[pl.ds(0, 128)],
                                agg_s.at[pl.ds(640 * s + 128 * k, 128)])
                return carry
            lax.fori_loop(0, ROWS_OUT, zcp, 0)

        def copy_out(dst_hbm):
            def ocp(k, carry):
                r = pl.ds(640 * s + 128 * k, 128)
                pltpu.sync_copy(agg_s.at[r], dst_hbm.at[c].at[r])
                return carry
            lax.fori_loop(0, ROWS_OUT, ocp, 0)

        fill(0.0)
        zero_agg()
        plsc.subcore_barrier()

        if with_deg:
            fill(1.0)

            def dchunk(g, carry):
                e0 = ((c * NS + s) * ROWS_PER_TILE + g * CH_ROWS) * 128
                pltpu.sync_copy(dstb.at[pl.ds(e0, nch)], dst_v)
                pltpu.sync_copy(rows_v, agg_s.at[dst_v], add=True)
                return carry
            lax.fori_loop(0, N_CHUNKS, dchunk, 0)
            plsc.subcore_barrier()
            copy_out(deg_out)
            plsc.subcore_barrier()
            fill(0.0)
            zero_agg()
            plsc.subcore_barrier()

        def chunk(g, carry):
            e0 = ((c * NS + s) * ROWS_PER_TILE + g * CH_ROWS) * 128
            pltpu.sync_copy(srcb.at[pl.ds(e0, nch)], src_v)
            pltpu.sync_copy(dstb.at[pl.ds(e0, nch)], dst_v)
            pltpu.async_copy(table.at[src_v], rows_v, gsem).wait()
            pltpu.sync_copy(rows_v, agg_s.at[dst_v], add=True)
            return carry
        lax.fori_loop(0, N_CHUNKS, chunk, 0)

        plsc.subcore_barrier()
        copy_out(agg_out)

    return pl.kernel(body, out_type=out_type, mesh=mesh,
                     scratch_types=scratch)


_sc_agg_deg = _make_sc_agg(with_deg=True)
_sc_agg = _make_sc_agg(with_deg=False)


def _tc_layer1(x, aggp, degp, ws, wn, b):
    def body(x_r, a0_r, a1_r, d0_r, d1_r, ws_r, wn_r, b_r, o_r, inv_r):
        deg = d0_r[0][:, 0:1] + d1_r[0][:, 0:1]
        inv = 1.0 / jnp.maximum(deg, 1.0)
        hn = (a0_r[0] + a1_r[0]) * inv
        h = (jnp.dot(x_r[...], ws_r[...], preferred_element_type=jnp.float32)
             + jnp.dot(hn, wn_r[...], preferred_element_type=jnp.float32)
             + b_r[...])
        o_r[...] = jnp.maximum(h, 0.0)
        inv_r[...] = inv

    return pl.pallas_call(
        body,
        grid=(10,),
        in_specs=[
            pl.BlockSpec((1000, D), lambda i: (i, 0)),
            pl.BlockSpec((1, 1000, D), lambda i: (0, i, 0)),
            pl.BlockSpec((1, 1000, D), lambda i: (1, i, 0)),
            pl.BlockSpec((1, 1000, D), lambda i: (0, i, 0)),
            pl.BlockSpec((1, 1000, D), lambda i: (1, i, 0)),
            pl.BlockSpec((D, D), lambda i: (0, 0)),
            pl.BlockSpec((D, D), lambda i: (0, 0)),
            pl.BlockSpec((1, D), lambda i: (0, 0)),
        ],
        out_specs=[
            pl.BlockSpec((1000, D), lambda i: (i, 0)),
            pl.BlockSpec((1000, 1), lambda i: (i, 0)),
        ],
        out_shape=[
            jax.ShapeDtypeStruct((N_NODES, D), jnp.float32),
            jax.ShapeDtypeStruct((N_NODES, 1), jnp.float32),
        ],
    )(x, aggp, aggp, degp, degp, ws, wn, b)


def _tc_layer2(h1, aggp, inv, ws, wn, b):
    def body(x_r, a0_r, a1_r, inv_r, ws_r, wn_r, b_r, o_r):
        hn = (a0_r[0] + a1_r[0]) * inv_r[...]
        o_r[...] = (
            jnp.dot(x_r[...], ws_r[...], preferred_element_type=jnp.float32)
            + jnp.dot(hn, wn_r[...], preferred_element_type=jnp.float32)
            + b_r[...])

    return pl.pallas_call(
        body,
        grid=(10,),
        in_specs=[
            pl.BlockSpec((1000, D), lambda i: (i, 0)),
            pl.BlockSpec((1, 1000, D), lambda i: (0, i, 0)),
            pl.BlockSpec((1, 1000, D), lambda i: (1, i, 0)),
            pl.BlockSpec((1000, 1), lambda i: (i, 0)),
            pl.BlockSpec((D, D), lambda i: (0, 0)),
            pl.BlockSpec((D, D), lambda i: (0, 0)),
            pl.BlockSpec((1, D), lambda i: (0, 0)),
        ],
        out_specs=pl.BlockSpec((1000, D), lambda i: (i, 0)),
        out_shape=jax.ShapeDtypeStruct((N_NODES, D), jnp.float32),
    )(h1, aggp, aggp, inv, ws, wn, b)


def kernel(x, edge_index, W_self1, W_neigh1, b1, W_self2, W_neigh2, b2):
    src = edge_index[0].astype(jnp.int32)
    dst = edge_index[1].astype(jnp.int32)
    npad = E_PAD - N_EDGES
    ar = jnp.arange(npad, dtype=jnp.int32)
    srcb = jnp.concatenate([src, ar % N_NODES])
    dstb = jnp.concatenate([dst, N_NODES + ar % (N_PAD - N_NODES)])
    b1r = b1.reshape(1, D)
    b2r = b2.reshape(1, D)

    agg1, degp = _sc_agg_deg(x, srcb, dstb)
    h1, inv = _tc_layer1(x, agg1, degp, W_self1, W_neigh1, b1r)
    agg2 = _sc_agg(h1, srcb, dstb)
    agg2 = agg2[0] if isinstance(agg2, (list, tuple)) else agg2
    return _tc_layer2(h1, agg2, inv, W_self2, W_neigh2, b2r)


# speedup vs baseline: 10.6825x; 1.8054x over previous
import jax
import jax.numpy as jnp
from jax import lax
from jax.experimental import pallas as pl
from jax.experimental.pallas import tpu as pltpu
from jax.experimental.pallas import tpu_sc as plsc

N_NODES = 10000
N_EDGES = 320000
D = 128
NS = 16
NC = 2
N_PAD = 10240
E_PAD = 327680
ROWS_PER_TILE = E_PAD // (NC * NS) // 128
BLK = 16
N_BLKS = ROWS_PER_TILE // BLK
ROWS_OUT = N_PAD // NS // 128


def _make_sc_agg(with_deg: bool):
    mesh = plsc.VectorSubcoreMesh(core_axis_name="c", subcore_axis_name="s")
    out_type = [
        jax.ShapeDtypeStruct((NC, N_PAD, D), jnp.float32),
    ]
    if with_deg:
        out_type.append(
            jax.ShapeDtypeStruct((NC, N_PAD, D), jnp.float32))
    scratch = [
        pltpu.VMEM((BLK, 128), jnp.int32),
        pltpu.VMEM((BLK, 128), jnp.int32),
        pltpu.VMEM((128, D), jnp.float32),
        pltpu.VMEM((128, D), jnp.float32),
        pltpu.VMEM_SHARED((N_PAD, D), jnp.float32),
        pltpu.SemaphoreType.DMA,
        pltpu.SemaphoreType.DMA,
    ]

    def body(table, srcb, dstb, *rest):
        if with_deg:
            (agg_out, deg_out,
             src_v, dst_v, rows0, rows1, agg_s, sem0, sem1) = rest
        else:
            (agg_out,
             src_v, dst_v, rows0, rows1, agg_s, sem0, sem1) = rest
        c = lax.axis_index("c")
        s = lax.axis_index("s")
        rows = (rows0, rows1)
        sems = (sem0, sem1)

        def fill(buf, val):
            v16 = jnp.full((16,), val, jnp.float32)

            def f(r, carry):
                for j in range(D // 16):
                    buf[r, pl.ds(j * 16, 16)] = v16
                return carry
            lax.fori_loop(0, 128, f, 0)

        def zero_agg():
            def zcp(k, carry):
                pltpu.sync_copy(rows0,
                                agg_s.at[pl.ds(640 * s + 128 * k, 128)])
                return carry
            lax.fori_loop(0, ROWS_OUT, zcp, 0)

        def copy_out(dst_hbm):
            def ocp(k, carry):
                r = pl.ds(640 * s + 128 * k, 128)
                pltpu.sync_copy(agg_s.at[r], dst_hbm.at[c].at[r])
                return carry
            lax.fori_loop(0, ROWS_OUT, ocp, 0)

        fill(rows0, 0.0)
        zero_agg()
        plsc.subcore_barrier()

        if with_deg:
            fill(rows0, 1.0)

            def dblk(t, carry):
                r0 = (c * NS + s) * ROWS_PER_TILE + t * BLK
                pltpu.sync_copy(dstb.at[pl.ds(r0, BLK)], dst_v)
                for k in range(BLK):
                    pltpu.sync_copy(rows0, agg_s.at[dst_v.at[k]], add=True)
                return carry
            lax.fori_loop(0, N_BLKS, dblk, 0)
            plsc.subcore_barrier()
            copy_out(deg_out)
            plsc.subcore_barrier()
            fill(rows0, 0.0)
            zero_agg()
            plsc.subcore_barrier()

        def ablk(t, carry):
            r0 = (c * NS + s) * ROWS_PER_TILE + t * BLK
            pltpu.sync_copy(srcb.at[pl.ds(r0, BLK)], src_v)
            pltpu.sync_copy(dstb.at[pl.ds(r0, BLK)], dst_v)
            cps = [pltpu.async_copy(table.at[src_v.at[0]], rows0, sem0)]
            for k in range(BLK):
                if k + 1 < BLK:
                    b = (k + 1) % 2
                    cps.append(pltpu.async_copy(
                        table.at[src_v.at[k + 1]], rows[b], sems[b]))
                cps[k].wait()
                pltpu.sync_copy(rows[k % 2], agg_s.at[dst_v.at[k]],
                                add=True)
            return carry
        lax.fori_loop(0, N_BLKS, ablk, 0)

        plsc.subcore_barrier()
        copy_out(agg_out)

    return pl.kernel(body, out_type=out_type, mesh=mesh,
                     scratch_types=scratch)


_sc_agg_deg = _make_sc_agg(with_deg=True)
_sc_agg = _make_sc_agg(with_deg=False)


def _tc_layer1(x, aggp, degp, ws, wn, b):
    def body(x_r, a0_r, a1_r, d0_r, d1_r, ws_r, wn_r, b_r, o_r, inv_r):
        deg = d0_r[0][:, 0:1] + d1_r[0][:, 0:1]
        inv = 1.0 / jnp.maximum(deg, 1.0)
        hn = (a0_r[0] + a1_r[0]) * inv
        h = (jnp.dot(x_r[...], ws_r[...], preferred_element_type=jnp.float32)
             + jnp.dot(hn, wn_r[...], preferred_element_type=jnp.float32)
             + b_r[...])
        o_r[...] = jnp.maximum(h, 0.0)
        inv_r[...] = inv

    return pl.pallas_call(
        body,
        grid=(10,),
        in_specs=[
            pl.BlockSpec((1000, D), lambda i: (i, 0)),
            pl.BlockSpec((1, 1000, D), lambda i: (0, i, 0)),
            pl.BlockSpec((1, 1000, D), lambda i: (1, i, 0)),
            pl.BlockSpec((1, 1000, D), lambda i: (0, i, 0)),
            pl.BlockSpec((1, 1000, D), lambda i: (1, i, 0)),
            pl.BlockSpec((D, D), lambda i: (0, 0)),
            pl.BlockSpec((D, D), lambda i: (0, 0)),
            pl.BlockSpec((1, D), lambda i: (0, 0)),
        ],
        out_specs=[
            pl.BlockSpec((1000, D), lambda i: (i, 0)),
            pl.BlockSpec((1000, 1), lambda i: (i, 0)),
        ],
        out_shape=[
            jax.ShapeDtypeStruct((N_NODES, D), jnp.float32),
            jax.ShapeDtypeStruct((N_NODES, 1), jnp.float32),
        ],
    )(x, aggp, aggp, degp, degp, ws, wn, b)


def _tc_layer2(h1, aggp, inv, ws, wn, b):
    def body(x_r, a0_r, a1_r, inv_r, ws_r, wn_r, b_r, o_r):
        hn = (a0_r[0] + a1_r[0]) * inv_r[...]
        o_r[...] = (
            jnp.dot(x_r[...], ws_r[...], preferred_element_type=jnp.float32)
            + jnp.dot(hn, wn_r[...], preferred_element_type=jnp.float32)
            + b_r[...])

    return pl.pallas_call(
        body,
        grid=(10,),
        in_specs=[
            pl.BlockSpec((1000, D), lambda i: (i, 0)),
            pl.BlockSpec((1, 1000, D), lambda i: (0, i, 0)),
            pl.BlockSpec((1, 1000, D), lambda i: (1, i, 0)),
            pl.BlockSpec((1000, 1), lambda i: (i, 0)),
            pl.BlockSpec((D, D), lambda i: (0, 0)),
            pl.BlockSpec((D, D), lambda i: (0, 0)),
            pl.BlockSpec((1, D), lambda i: (0, 0)),
        ],
        out_specs=pl.BlockSpec((1000, D), lambda i: (i, 0)),
        out_shape=jax.ShapeDtypeStruct((N_NODES, D), jnp.float32),
    )(h1, aggp, aggp, inv, ws, wn, b)


def kernel(x, edge_index, W_self1, W_neigh1, b1, W_self2, W_neigh2, b2):
    src = edge_index[0].astype(jnp.int32)
    dst = edge_index[1].astype(jnp.int32)
    npad = E_PAD - N_EDGES
    ar = jnp.arange(npad, dtype=jnp.int32)
    srcb = jnp.concatenate([src, ar % N_NODES]).reshape(E_PAD // 128, 128)
    dstb = jnp.concatenate(
        [dst, N_NODES + ar % (N_PAD - N_NODES)]).reshape(E_PAD // 128, 128)
    b1r = b1.reshape(1, D)
    b2r = b2.reshape(1, D)

    agg1, degp = _sc_agg_deg(x, srcb, dstb)
    h1, inv = _tc_layer1(x, agg1, degp, W_self1, W_neigh1, b1r)
    agg2 = _sc_agg(h1, srcb, dstb)
    agg2 = agg2[0] if isinstance(agg2, (list, tuple)) else agg2
    return _tc_layer2(h1, agg2, inv, W_self2, W_neigh2, b2r)
